# baseline (speedup 1.0000x reference)
import functools

import jax
import jax.numpy as jnp
from jax import lax
from jax.experimental import pallas as pl
from jax.experimental.pallas import tpu as pltpu
from jax.experimental.pallas import tpu_sc as plsc

N = 10000
D = 128
H = 64
E = 320000

NP = 10240
NC = 2
NS = 16
NW = NC * NS
EB = 128
EPT = 10240
NB = EPT // EB
CB = 40
EP = NW * EPT
RPS = NP // NS


@functools.cache
def _mesh():
    return plsc.VectorSubcoreMesh(core_axis_name="c", subcore_axis_name="s",
                                  num_cores=NC, num_subcores=NS)


@functools.cache
def _make_deg():
    return functools.partial(
        pl.kernel,
        out_type=jax.ShapeDtypeStruct((NC, NP), jnp.float32),
        mesh=_mesh(),
        scratch_types=[
            pltpu.VMEM((NB, EB), jnp.int32),
            pltpu.VMEM((NB, EB), jnp.float32),
            pltpu.VMEM_SHARED((NP,), jnp.float32),
        ],
        compiler_params=pltpu.CompilerParams(
            disable_bounds_checks=True, disable_semaphore_checks=True),
    )(_deg_body)


def _deg_body(dst_hbm, ones_hbm, zeros_hbm, deg_out, dst_v, ones_v, deg_sh):
    c = lax.axis_index("c")
    s = lax.axis_index("s")
    w = s * NC + c
    pltpu.sync_copy(dst_hbm.at[w], dst_v)
    pltpu.sync_copy(ones_hbm, ones_v)
    pltpu.sync_copy(zeros_hbm, deg_sh.at[pl.ds(s * RPS, RPS)])
    plsc.subcore_barrier()

    def body(j, carry):
        pltpu.sync_copy(ones_v.at[j], deg_sh.at[dst_v.at[j]], add=True)
        return carry

    lax.fori_loop(0, NB, body, 0)
    plsc.subcore_barrier()
    pltpu.sync_copy(deg_sh.at[pl.ds(s * RPS, RPS)],
                    deg_out.at[c, pl.ds(s * RPS, RPS)])


@functools.cache
def _make_agg(F):
    def _agg(hs_hbm, src_hbm, dst_hbm, zeros_hbm, acc_out,
             src_c, dst_c, rows0, rows1, acc_sh, gsem0, gsem1):
        c = lax.axis_index("c")
        s = lax.axis_index("s")
        w = s * NC + c
        pltpu.sync_copy(zeros_hbm, acc_sh.at[pl.ds(s * RPS, RPS)])
        plsc.subcore_barrier()

        def chunk(ck, carry):
            pltpu.sync_copy(src_hbm.at[w, pl.ds(ck * CB, CB)], src_c)
            pltpu.sync_copy(dst_hbm.at[w, pl.ds(ck * CB, CB)], dst_c)
            pltpu.async_copy(hs_hbm.at[src_c.at[0]], rows0, gsem0)

            def body(t, carry2):
                j0 = 2 * t
                j1 = j0 + 1
                pltpu.async_copy(hs_hbm.at[src_c.at[j1]], rows1, gsem1)
                pltpu.make_async_copy(hs_hbm.at[src_c.at[j0]], rows0,
                                      gsem0).wait()
                pltpu.sync_copy(rows0, acc_sh.at[dst_c.at[j0]], add=True)

                @pl.when(j1 + 1 < CB)
                def _():
                    pltpu.async_copy(hs_hbm.at[src_c.at[j1 + 1]], rows0,
                                     gsem0)

                pltpu.make_async_copy(hs_hbm.at[src_c.at[j1]], rows1,
                                      gsem1).wait()
                pltpu.sync_copy(rows1, acc_sh.at[dst_c.at[j1]], add=True)
                return carry2

            lax.fori_loop(0, CB // 2, body, 0, unroll=2)
            return carry

        lax.fori_loop(0, NB // CB, chunk, 0)
        plsc.subcore_barrier()
        pltpu.sync_copy(acc_sh.at[pl.ds(s * RPS, RPS)],
                        acc_out.at[c, pl.ds(s * RPS, RPS)])

    return functools.partial(
        pl.kernel,
        out_type=jax.ShapeDtypeStruct((NC, NP, F), jnp.float32),
        mesh=_mesh(),
        scratch_types=[
            pltpu.VMEM((CB, EB), jnp.int32),
            pltpu.VMEM((CB, EB), jnp.int32),
            pltpu.VMEM((EB, D), jnp.float32),
            pltpu.VMEM((EB, D), jnp.float32),
            pltpu.VMEM_SHARED((NP, F), jnp.float32),
            pltpu.SemaphoreType.DMA,
            pltpu.SemaphoreType.DMA,
        ],
        compiler_params=pltpu.CompilerParams(
            disable_bounds_checks=True, disable_semaphore_checks=True),
    )(_agg)


BR = 5120
GR = NP // BR


def _mm1a_body(emb_ref, w1_ref, h1_ref):
    h1_ref[...] = jnp.dot(emb_ref[...], w1_ref[...],
                          preferred_element_type=jnp.float32)


_mm1a = pl.pallas_call(
    _mm1a_body,
    grid=(GR,),
    in_specs=[
        pl.BlockSpec((BR, D), lambda i: (i, 0)),
        pl.BlockSpec((D, H), lambda i: (0, 0)),
    ],
    out_specs=pl.BlockSpec((BR, H), lambda i: (i, 0)),
    out_shape=jax.ShapeDtypeStruct((NP, H), jnp.float32),
)


def _mm1b_body(h1_ref, degp_ref, hs1_ref, dinv_ref):
    deg = degp_ref[0] + degp_ref[1] + 1.0
    dinv = lax.rsqrt(deg)
    h = h1_ref[...]
    hs1_ref[...] = jnp.concatenate([h * dinv, jnp.zeros_like(h)], axis=1)
    dinv_ref[...] = dinv


_mm1b = pl.pallas_call(
    _mm1b_body,
    grid=(GR,),
    in_specs=[
        pl.BlockSpec((BR, H), lambda i: (i, 0)),
        pl.BlockSpec((2, BR, 1), lambda i: (0, i, 0)),
    ],
    out_specs=[
        pl.BlockSpec((BR, D), lambda i: (i, 0)),
        pl.BlockSpec((BR, 1), lambda i: (i, 0)),
    ],
    out_shape=[
        jax.ShapeDtypeStruct((NP, D), jnp.float32),
        jax.ShapeDtypeStruct((NP, 1), jnp.float32),
    ],
)


def _mm2_body(acc_ref, hs1_ref, dinv_ref, b1_ref, w2_ref, hs2_ref):
    agg = acc_ref[0, :, :H] + acc_ref[1, :, :H] + hs1_ref[:, :H]
    o1 = jnp.maximum(agg * dinv_ref[...] + b1_ref[...], 0.0)
    h2 = jnp.dot(o1, w2_ref[...], preferred_element_type=jnp.float32)
    hs2_ref[...] = h2 * dinv_ref[...]


_mm2 = pl.pallas_call(
    _mm2_body,
    grid=(GR,),
    in_specs=[
        pl.BlockSpec((2, BR, D), lambda i: (0, i, 0)),
        pl.BlockSpec((BR, D), lambda i: (i, 0)),
        pl.BlockSpec((BR, 1), lambda i: (i, 0)),
        pl.BlockSpec((1, H), lambda i: (0, 0)),
        pl.BlockSpec((H, D), lambda i: (0, 0)),
    ],
    out_specs=pl.BlockSpec((BR, D), lambda i: (i, 0)),
    out_shape=jax.ShapeDtypeStruct((NP, D), jnp.float32),
)


def _fin_body(acc_ref, hs2_ref, dinv_ref, b2_ref, out_ref):
    agg = acc_ref[0] + acc_ref[1] + hs2_ref[...]
    out_ref[...] = jnp.maximum(agg * dinv_ref[...] + b2_ref[...], 0.0)


_fin = pl.pallas_call(
    _fin_body,
    grid=(GR,),
    in_specs=[
        pl.BlockSpec((2, BR, D), lambda i: (0, i, 0)),
        pl.BlockSpec((BR, D), lambda i: (i, 0)),
        pl.BlockSpec((BR, 1), lambda i: (i, 0)),
        pl.BlockSpec((1, D), lambda i: (0, 0)),
    ],
    out_specs=pl.BlockSpec((BR, D), lambda i: (i, 0)),
    out_shape=jax.ShapeDtypeStruct((N, D), jnp.float32),
)


def kernel(embedding, edge_index, W1, b1, W2, b2):
    ei = edge_index.astype(jnp.int32)
    pad_idx = N + (jnp.arange(EP - E, dtype=jnp.int32) % (NP - N))
    src = jnp.concatenate([ei[0], pad_idx]).reshape(NW, NB, EB)
    dst = jnp.concatenate([ei[1], pad_idx]).reshape(NW, NB, EB)
    embp = jnp.pad(embedding, ((0, NP - N), (0, 0)))
    zd = jnp.zeros((RPS,), jnp.float32)
    z2 = jnp.zeros((RPS, D), jnp.float32)

    ones = jnp.ones((NB, EB), jnp.float32)
    h1 = _mm1a(embp, W1)
    degp = _make_deg()(dst, ones, zd)
    hs1, dinv = _mm1b(h1, degp.reshape(NC, NP, 1))
    acc1 = _make_agg(D)(hs1, src, dst, z2)
    hs2 = _mm2(acc1, hs1, dinv, b1.reshape(1, H), W2)
    acc2 = _make_agg(D)(hs2, src, dst, z2)
    return _fin(acc2, hs2, dinv, b2.reshape(1, D))


# speedup vs baseline: 1.0164x; 1.0164x over previous
import functools

import jax
import jax.numpy as jnp
from jax import lax
from jax.experimental import pallas as pl
from jax.experimental.pallas import tpu as pltpu
from jax.experimental.pallas import tpu_sc as plsc

N = 10000
D = 128
H = 64
E = 320000

NP = 10240
NC = 2
NS = 16
NW = NC * NS
EB = 128
EPT = 10240
NB = EPT // EB
CB = 40
EP = NW * EPT
RPS = NP // NS


@functools.cache
def _mesh():
    return plsc.VectorSubcoreMesh(core_axis_name="c", subcore_axis_name="s",
                                  num_cores=NC, num_subcores=NS)


@functools.cache
def _make_deg():
    return functools.partial(
        pl.kernel,
        out_type=jax.ShapeDtypeStruct((NC, NP), jnp.float32),
        mesh=_mesh(),
        scratch_types=[
            pltpu.VMEM((NB, EB), jnp.int32),
            pltpu.VMEM((EB,), jnp.float32),
            pltpu.VMEM_SHARED((NP,), jnp.float32),
        ],
        compiler_params=pltpu.CompilerParams(
            disable_bounds_checks=True, disable_semaphore_checks=True),
    )(_deg_body)


def _deg_body(dst_hbm, ones_hbm, zeros_hbm, deg_out, dst_v, ones_v, deg_sh):
    c = lax.axis_index("c")
    s = lax.axis_index("s")
    w = s * NC + c
    pltpu.sync_copy(dst_hbm.at[w], dst_v)
    pltpu.sync_copy(ones_hbm, ones_v)
    pltpu.sync_copy(zeros_hbm, deg_sh.at[pl.ds(s * RPS, RPS)])
    plsc.subcore_barrier()

    def body(j, carry):
        pltpu.sync_copy(ones_v, deg_sh.at[dst_v.at[j]], add=True)
        return carry

    lax.fori_loop(0, NB, body, 0)
    plsc.subcore_barrier()
    pltpu.sync_copy(deg_sh.at[pl.ds(s * RPS, RPS)],
                    deg_out.at[c, pl.ds(s * RPS, RPS)])


@functools.cache
def _make_agg(F):
    def _agg(hs_hbm, src_hbm, dst_hbm, zeros_hbm, acc_out,
             src_c, dst_c, rows0, rows1, acc_sh, gsem0, gsem1):
        c = lax.axis_index("c")
        s = lax.axis_index("s")
        w = s * NC + c
        pltpu.sync_copy(zeros_hbm, acc_sh.at[pl.ds(s * RPS, RPS)])
        plsc.subcore_barrier()

        def chunk(ck, carry):
            pltpu.sync_copy(src_hbm.at[w, pl.ds(ck * CB, CB)], src_c)
            pltpu.sync_copy(dst_hbm.at[w, pl.ds(ck * CB, CB)], dst_c)
            pltpu.async_copy(hs_hbm.at[src_c.at[0]], rows0, gsem0)

            def body(t, carry2):
                j0 = 2 * t
                j1 = j0 + 1
                pltpu.async_copy(hs_hbm.at[src_c.at[j1]], rows1, gsem1)
                pltpu.make_async_copy(hs_hbm.at[src_c.at[j0]], rows0,
                                      gsem0).wait()
                pltpu.sync_copy(rows0, acc_sh.at[dst_c.at[j0]], add=True)

                @pl.when(j1 + 1 < CB)
                def _():
                    pltpu.async_copy(hs_hbm.at[src_c.at[j1 + 1]], rows0,
                                     gsem0)

                pltpu.make_async_copy(hs_hbm.at[src_c.at[j1]], rows1,
                                      gsem1).wait()
                pltpu.sync_copy(rows1, acc_sh.at[dst_c.at[j1]], add=True)
                return carry2

            lax.fori_loop(0, CB // 2, body, 0, unroll=2)
            return carry

        lax.fori_loop(0, NB // CB, chunk, 0)
        plsc.subcore_barrier()
        pltpu.sync_copy(acc_sh.at[pl.ds(s * RPS, RPS)],
                        acc_out.at[c, pl.ds(s * RPS, RPS)])

    return functools.partial(
        pl.kernel,
        out_type=jax.ShapeDtypeStruct((NC, NP, F), jnp.float32),
        mesh=_mesh(),
        scratch_types=[
            pltpu.VMEM((CB, EB), jnp.int32),
            pltpu.VMEM((CB, EB), jnp.int32),
            pltpu.VMEM((EB, D), jnp.float32),
            pltpu.VMEM((EB, D), jnp.float32),
            pltpu.VMEM_SHARED((NP, F), jnp.float32),
            pltpu.SemaphoreType.DMA,
            pltpu.SemaphoreType.DMA,
        ],
        compiler_params=pltpu.CompilerParams(
            disable_bounds_checks=True, disable_semaphore_checks=True),
    )(_agg)


BR = 5120
GR = NP // BR


def _mm1a_body(emb_ref, w1_ref, h1_ref):
    h1_ref[...] = jnp.dot(emb_ref[...], w1_ref[...],
                          preferred_element_type=jnp.float32)


_mm1a = pl.pallas_call(
    _mm1a_body,
    grid=(GR,),
    in_specs=[
        pl.BlockSpec((BR, D), lambda i: (i, 0)),
        pl.BlockSpec((D, H), lambda i: (0, 0)),
    ],
    out_specs=pl.BlockSpec((BR, H), lambda i: (i, 0)),
    out_shape=jax.ShapeDtypeStruct((NP, H), jnp.float32),
)


def _mm1b_body(h1_ref, degp_ref, hs1_ref, dinv_ref):
    deg = degp_ref[0] + degp_ref[1] + 1.0
    dinv = lax.rsqrt(deg)
    h = h1_ref[...]
    hs1_ref[...] = jnp.concatenate([h * dinv, jnp.zeros_like(h)], axis=1)
    dinv_ref[...] = dinv


_mm1b = pl.pallas_call(
    _mm1b_body,
    grid=(GR,),
    in_specs=[
        pl.BlockSpec((BR, H), lambda i: (i, 0)),
        pl.BlockSpec((2, BR, 1), lambda i: (0, i, 0)),
    ],
    out_specs=[
        pl.BlockSpec((BR, D), lambda i: (i, 0)),
        pl.BlockSpec((BR, 1), lambda i: (i, 0)),
    ],
    out_shape=[
        jax.ShapeDtypeStruct((NP, D), jnp.float32),
        jax.ShapeDtypeStruct((NP, 1), jnp.float32),
    ],
)


def _mm2_body(acc_ref, hs1_ref, dinv_ref, b1_ref, w2_ref, hs2_ref):
    agg = acc_ref[0, :, :H] + acc_ref[1, :, :H] + hs1_ref[:, :H]
    o1 = jnp.maximum(agg * dinv_ref[...] + b1_ref[...], 0.0)
    h2 = jnp.dot(o1, w2_ref[...], preferred_element_type=jnp.float32)
    hs2_ref[...] = h2 * dinv_ref[...]


_mm2 = pl.pallas_call(
    _mm2_body,
    grid=(GR,),
    in_specs=[
        pl.BlockSpec((2, BR, D), lambda i: (0, i, 0)),
        pl.BlockSpec((BR, D), lambda i: (i, 0)),
        pl.BlockSpec((BR, 1), lambda i: (i, 0)),
        pl.BlockSpec((1, H), lambda i: (0, 0)),
        pl.BlockSpec((H, D), lambda i: (0, 0)),
    ],
    out_specs=pl.BlockSpec((BR, D), lambda i: (i, 0)),
    out_shape=jax.ShapeDtypeStruct((NP, D), jnp.float32),
)


def _fin_body(acc_ref, hs2_ref, dinv_ref, b2_ref, out_ref):
    agg = acc_ref[0] + acc_ref[1] + hs2_ref[...]
    out_ref[...] = jnp.maximum(agg * dinv_ref[...] + b2_ref[...], 0.0)


_fin = pl.pallas_call(
    _fin_body,
    grid=(GR,),
    in_specs=[
        pl.BlockSpec((2, BR, D), lambda i: (0, i, 0)),
        pl.BlockSpec((BR, D), lambda i: (i, 0)),
        pl.BlockSpec((BR, 1), lambda i: (i, 0)),
        pl.BlockSpec((1, D), lambda i: (0, 0)),
    ],
    out_specs=pl.BlockSpec((BR, D), lambda i: (i, 0)),
    out_shape=jax.ShapeDtypeStruct((N, D), jnp.float32),
)


def kernel(embedding, edge_index, W1, b1, W2, b2):
    ei = edge_index.astype(jnp.int32)
    pad_idx = N + (jnp.arange(EP - E, dtype=jnp.int32) % (NP - N))
    src = jnp.concatenate([ei[0], pad_idx]).reshape(NW, NB, EB)
    dst = jnp.concatenate([ei[1], pad_idx]).reshape(NW, NB, EB)
    zd = jnp.zeros((RPS,), jnp.float32)
    z2 = jnp.zeros((RPS, D), jnp.float32)

    ones = jnp.ones((EB,), jnp.float32)
    h1 = _mm1a(embedding, W1)
    degp = _make_deg()(dst, ones, zd)
    hs1, dinv = _mm1b(h1, degp.reshape(NC, NP, 1))
    acc1 = _make_agg(D)(hs1, src, dst, z2)
    hs2 = _mm2(acc1, hs1, dinv, b1.reshape(1, H), W2)
    acc2 = _make_agg(D)(hs2, src, dst, z2)
    return _fin(acc2, hs2, dinv, b2.reshape(1, D))
